# speedup vs baseline: 4.2662x; 4.2662x over previous
import functools

import jax
import jax.numpy as jnp
from jax import lax
from jax.experimental import pallas as pl
from jax.experimental.pallas import tpu as pltpu
from jax.experimental.pallas import tpu_sc as plsc

N_ROWS = 100000
D = 128
NSEG = 64
CHUNK = 128
N_FULL = N_ROWS // CHUNK
TAIL = N_ROWS - N_FULL * CHUNK
NC, NS = 2, 16
NW = NC * NS
ITERS = (N_FULL + NW) // NW
TAIL_W = N_FULL % NW


def _body(nodes, ids, zeros, out, idx_v, tidx_v, chunk_v, tail_v, acc_sh):
    c = lax.axis_index("c")
    s = lax.axis_index("s")
    wid = s * NC + c

    @pl.when(s == 0)
    def _init():
        pltpu.sync_copy(zeros, acc_sh)

    plsc.subcore_barrier()

    def step(i, carry):
        ch = wid + i * NW

        @pl.when(ch < N_FULL)
        def _full():
            r0 = ch * CHUNK
            pltpu.sync_copy(ids.at[pl.ds(r0, CHUNK)], idx_v)
            pltpu.sync_copy(nodes.at[pl.ds(r0, CHUNK)], chunk_v)
            pltpu.sync_copy(chunk_v, acc_sh.at[idx_v], add=True)

        return carry

    lax.fori_loop(0, ITERS, step, 0)

    @pl.when(wid == TAIL_W)
    def _tail():
        r0 = N_FULL * CHUNK
        pltpu.sync_copy(ids.at[pl.ds(r0, TAIL)], tidx_v)
        pltpu.sync_copy(nodes.at[pl.ds(r0, TAIL)], tail_v)
        pltpu.sync_copy(tail_v, acc_sh.at[tidx_v], add=True)

    plsc.subcore_barrier()

    @pl.when(s == 0)
    def _flush():
        pltpu.sync_copy(acc_sh, out.at[c])


@jax.jit
def _segsum(nodes, ids32, zeros):
    mesh = plsc.VectorSubcoreMesh(core_axis_name="c", subcore_axis_name="s")
    partials = pl.kernel(
        _body,
        out_type=jax.ShapeDtypeStruct((NC, NSEG, D), jnp.float32),
        mesh=mesh,
        scratch_types=[
            pltpu.VMEM((CHUNK,), jnp.int32),
            pltpu.VMEM((TAIL,), jnp.int32),
            pltpu.VMEM((CHUNK, D), jnp.float32),
            pltpu.VMEM((TAIL, D), jnp.float32),
            pltpu.VMEM_SHARED((NSEG, D), jnp.float32),
        ],
    )(nodes, ids32, zeros)
    return partials[0] + partials[1]


def kernel(nodes, segment_ids, num_graphs):
    del num_graphs
    ids32 = segment_ids.astype(jnp.int32)
    zeros = jnp.zeros((NSEG, D), jnp.float32)
    return _segsum(nodes, ids32, zeros)


# speedup vs baseline: 6.3277x; 1.4832x over previous
import jax
import jax.numpy as jnp
from jax import lax
from jax.experimental import pallas as pl
from jax.experimental.pallas import tpu as pltpu
from jax.experimental.pallas import tpu_sc as plsc

N_ROWS = 100000
D = 128
NSEG = 64
CHUNK = 128
N_FULL = N_ROWS // CHUNK
TAIL = N_ROWS - N_FULL * CHUNK
NC, NS = 2, 16
NW = NC * NS
MAXC = -(-N_FULL // NW)
HI = N_FULL - (MAXC - 1) * NW


def _body(nodes, ids, zeros, out,
          ibuf0, ibuf1, tidx_v, buf0, buf1, tail_v, acc_sh,
          semn0, semn1, semi0, semi1, sem_t):
    c = lax.axis_index("c")
    s = lax.axis_index("s")
    wid = s * NC + c

    bufs = (buf0, buf1)
    ibufs = (ibuf0, ibuf1)
    semns = (semn0, semn1)
    semis = (semi0, semi1)

    def gather(j):
        r0 = (wid + j * NW) * CHUNK
        pltpu.async_copy(nodes.at[pl.ds(r0, CHUNK)], bufs[j % 2], semns[j % 2])
        pltpu.async_copy(ids.at[pl.ds(r0, CHUNK)], ibufs[j % 2], semis[j % 2])

    def gather_wait(j):
        pltpu.make_async_copy(nodes.at[pl.ds(0, CHUNK)], bufs[j % 2], semns[j % 2]).wait()
        pltpu.make_async_copy(ids.at[pl.ds(0, CHUNK)], ibufs[j % 2], semis[j % 2]).wait()

    gather(0)
    gather(1)

    @pl.when(s == 0)
    def _init():
        pltpu.sync_copy(zeros, acc_sh)

    plsc.subcore_barrier()

    for i in range(MAXC):
        def step(i=i):
            gather_wait(i)
            pltpu.sync_copy(bufs[i % 2], acc_sh.at[ibufs[i % 2]], add=True)
            j = i + 2
            if j < MAXC - 1:
                gather(j)
            elif j == MAXC - 1:
                @pl.when(wid < HI)
                def _():
                    gather(j)

        if i < MAXC - 1:
            step()
        else:
            @pl.when(wid < HI)
            def _():
                step()

    @pl.when(wid == NW - 1)
    def _tail():
        r0 = N_FULL * CHUNK
        pltpu.sync_copy(ids.at[pl.ds(r0, TAIL)], tidx_v)
        pltpu.async_copy(nodes.at[pl.ds(r0, TAIL)], tail_v, sem_t).wait()
        pltpu.sync_copy(tail_v, acc_sh.at[tidx_v], add=True)

    plsc.subcore_barrier()

    @pl.when(s == 0)
    def _flush():
        pltpu.sync_copy(acc_sh, out.at[c])


@jax.jit
def _segsum(nodes, ids32, zeros):
    mesh = plsc.VectorSubcoreMesh(core_axis_name="c", subcore_axis_name="s")
    partials = pl.kernel(
        _body,
        out_type=jax.ShapeDtypeStruct((NC, NSEG, D), jnp.float32),
        mesh=mesh,
        scratch_types=[
            pltpu.VMEM((CHUNK,), jnp.int32),
            pltpu.VMEM((CHUNK,), jnp.int32),
            pltpu.VMEM((TAIL,), jnp.int32),
            pltpu.VMEM((CHUNK, D), jnp.float32),
            pltpu.VMEM((CHUNK, D), jnp.float32),
            pltpu.VMEM((TAIL, D), jnp.float32),
            pltpu.VMEM_SHARED((NSEG, D), jnp.float32),
            pltpu.SemaphoreType.DMA,
            pltpu.SemaphoreType.DMA,
            pltpu.SemaphoreType.DMA,
            pltpu.SemaphoreType.DMA,
            pltpu.SemaphoreType.DMA,
        ],
    )(nodes, ids32, zeros)
    return partials[0] + partials[1]


def kernel(nodes, segment_ids, num_graphs):
    del num_graphs
    ids32 = segment_ids.astype(jnp.int32)
    zeros = jnp.zeros((NSEG, D), jnp.float32)
    return _segsum(nodes, ids32, zeros)


# speedup vs baseline: 6.4447x; 1.0185x over previous
import jax
import jax.numpy as jnp
from jax import lax
from jax.experimental import pallas as pl
from jax.experimental.pallas import tpu as pltpu
from jax.experimental.pallas import tpu_sc as plsc

N_ROWS = 100000
D = 128
NSEG = 64
CHUNK = 128
N_FULL = N_ROWS // CHUNK
TAIL = N_ROWS - N_FULL * CHUNK
NC, NS = 2, 16
NW = NC * NS
MAXC = -(-N_FULL // NW)
HI = N_FULL - (MAXC - 1) * NW
NBUF = 3


def _body(nodes, ids, zeros, out,
          ibufs, bufs, tidx_v, tail_v, acc_sh, semns, semis, semscs, sem_t):
    c = lax.axis_index("c")
    s = lax.axis_index("s")
    wid = s * NC + c

    def gather(j):
        b = j % NBUF
        r0 = (wid + j * NW) * CHUNK
        pltpu.async_copy(nodes.at[pl.ds(r0, CHUNK)], bufs[b], semns[b])
        pltpu.async_copy(ids.at[pl.ds(r0, CHUNK)], ibufs[b], semis[b])

    def gather_wait(j):
        b = j % NBUF
        pltpu.make_async_copy(nodes.at[pl.ds(0, CHUNK)], bufs[b], semns[b]).wait()
        pltpu.make_async_copy(ids.at[pl.ds(0, CHUNK)], ibufs[b], semis[b]).wait()

    def scatter(j):
        b = j % NBUF
        pltpu.async_copy(bufs[b], acc_sh.at[ibufs[b]], semscs[b], add=True)

    def scatter_wait(j):
        b = j % NBUF
        pltpu.make_async_copy(bufs[b], acc_sh.at[ibufs[b]], semscs[b]).wait()

    gather(0)

    @pl.when(s == 0)
    def _init():
        pltpu.sync_copy(zeros, acc_sh)

    plsc.subcore_barrier()

    for i in range(MAXC):
        if i >= 2:
            scatter_wait(i - 2)

        def step(i=i):
            j = i + 1
            if j < MAXC:
                if j == MAXC - 1:
                    @pl.when(wid < HI)
                    def _():
                        gather(j)
                else:
                    gather(j)
            gather_wait(i)
            scatter(i)

        if i == MAXC - 1:
            @pl.when(wid < HI)
            def _():
                step()
        else:
            step()

    scatter_wait(MAXC - 2)

    @pl.when(wid < HI)
    def _last():
        scatter_wait(MAXC - 1)

    @pl.when(wid == NW - 1)
    def _tail():
        r0 = N_FULL * CHUNK
        pltpu.sync_copy(ids.at[pl.ds(r0, TAIL)], tidx_v)
        pltpu.async_copy(nodes.at[pl.ds(r0, TAIL)], tail_v, sem_t).wait()
        pltpu.sync_copy(tail_v, acc_sh.at[tidx_v], add=True)

    plsc.subcore_barrier()

    @pl.when(s == 0)
    def _flush():
        pltpu.sync_copy(acc_sh, out.at[c])


@jax.jit
def _segsum(nodes, ids32, zeros):
    mesh = plsc.VectorSubcoreMesh(core_axis_name="c", subcore_axis_name="s")
    partials = pl.kernel(
        _body,
        out_type=jax.ShapeDtypeStruct((NC, NSEG, D), jnp.float32),
        mesh=mesh,
        scratch_types=[
            [pltpu.VMEM((CHUNK,), jnp.int32) for _ in range(NBUF)],
            [pltpu.VMEM((CHUNK, D), jnp.float32) for _ in range(NBUF)],
            pltpu.VMEM((TAIL,), jnp.int32),
            pltpu.VMEM((TAIL, D), jnp.float32),
            pltpu.VMEM_SHARED((NSEG, D), jnp.float32),
            [pltpu.SemaphoreType.DMA for _ in range(NBUF)],
            [pltpu.SemaphoreType.DMA for _ in range(NBUF)],
            [pltpu.SemaphoreType.DMA for _ in range(NBUF)],
            pltpu.SemaphoreType.DMA,
        ],
    )(nodes, ids32, zeros)
    return partials[0] + partials[1]


def kernel(nodes, segment_ids, num_graphs):
    del num_graphs
    ids32 = segment_ids.astype(jnp.int32)
    zeros = jnp.zeros((NSEG, D), jnp.float32)
    return _segsum(nodes, ids32, zeros)
